# baseline (speedup 1.0000x reference)
import functools

import jax
import jax.numpy as jnp
from jax import lax
from jax.experimental import pallas as pl
from jax.experimental.pallas import tpu as pltpu
from jax.experimental.pallas import tpu_sc as plsc

N_NODES = 10000
NNZ = 320000
D_FEAT = 128

NUM_CORES = 2
NUM_SUBCORES = 16
NW = NUM_CORES * NUM_SUBCORES
PER_W = NNZ // NW
CHUNK = 128
N_FULL_CHUNKS = PER_W // CHUNK
TAIL_ROWS = PER_W - N_FULL_CHUNKS * CHUNK
IDX_ROWS = N_FULL_CHUNKS + 2
NBUF = 2
N_ROUNDS = N_FULL_CHUNKS // NBUF - 1
STAGE_ROWS = 624
STAGE_LAST = N_NODES - 15 * STAGE_ROWS


@functools.partial(
    pl.kernel,
    out_type=jax.ShapeDtypeStruct((NNZ, D_FEAT), jnp.float32),
    mesh=plsc.VectorSubcoreMesh(core_axis_name="c", subcore_axis_name="s"),
    scratch_types=[
        pltpu.VMEM_SHARED((N_NODES, D_FEAT), jnp.float32),
        pltpu.VMEM((IDX_ROWS, CHUNK), jnp.int32),
        pltpu.VMEM((CHUNK, D_FEAT), jnp.float32),
        pltpu.VMEM((CHUNK, D_FEAT), jnp.float32),
        pltpu.SemaphoreType.DMA,
        pltpu.SemaphoreType.DMA,
    ],
)
def _unpool(x_hbm, idx_hbm, out_hbm, x_spmem, idx_v, r0, r1, s0, s1):
    bufs = (r0, r1)
    sems = (s0, s1)
    sid = lax.axis_index("s").astype(jnp.int32)
    wid = sid * jnp.int32(NUM_CORES) + lax.axis_index("c").astype(jnp.int32)
    base = wid * jnp.int32(PER_W)

    xoff = sid * jnp.int32(STAGE_ROWS)

    @pl.when(sid < jnp.int32(NUM_SUBCORES - 1))
    def _():
        pltpu.sync_copy(
            x_hbm.at[pl.ds(xoff, STAGE_ROWS)],
            x_spmem.at[pl.ds(xoff, STAGE_ROWS)],
        )

    @pl.when(sid == jnp.int32(NUM_SUBCORES - 1))
    def _():
        pltpu.sync_copy(
            x_hbm.at[pl.ds(xoff, STAGE_LAST)],
            x_spmem.at[pl.ds(xoff, STAGE_LAST)],
        )

    pltpu.sync_copy(idx_hbm.at[wid], idx_v)
    plsc.subcore_barrier()

    def start_gather(c, b):
        pltpu.async_copy(x_spmem.at[idx_v.at[c]], bufs[b], sems[b])

    def wait_buf_dma(b):
        pltpu.make_async_copy(
            out_hbm.at[pl.ds(jnp.int32(0), CHUNK)], bufs[b], sems[b]
        ).wait()

    def start_store(c, b):
        off = base + c * jnp.int32(CHUNK)
        pltpu.async_copy(bufs[b], out_hbm.at[pl.ds(off, CHUNK)], sems[b])

    for b in range(NBUF):
        start_gather(jnp.int32(b), b)
    for b in range(NBUF):
        wait_buf_dma(b)
        start_store(jnp.int32(b), b)

    def body(g, _):
        for b in range(NBUF):
            c = g * jnp.int32(NBUF) + jnp.int32(b)
            wait_buf_dma(b)
            start_gather(c, b)
        for b in range(NBUF):
            c = g * jnp.int32(NBUF) + jnp.int32(b)
            wait_buf_dma(b)
            start_store(c, b)
        return ()

    lax.fori_loop(
        jnp.int32(1), jnp.int32(N_ROUNDS + 1), body, (), unroll=False
    )

    wait_buf_dma(0)
    start_gather(jnp.int32(N_FULL_CHUNKS), 0)
    wait_buf_dma(0)
    tail_off = base + jnp.int32(N_FULL_CHUNKS * CHUNK)
    pltpu.async_copy(
        bufs[0].at[pl.ds(jnp.int32(0), TAIL_ROWS)],
        out_hbm.at[pl.ds(tail_off, TAIL_ROWS)],
        sems[0],
    )
    pltpu.make_async_copy(
        bufs[0].at[pl.ds(jnp.int32(0), TAIL_ROWS)],
        out_hbm.at[pl.ds(tail_off, TAIL_ROWS)],
        sems[0],
    ).wait()
    wait_buf_dma(1)


def kernel(X, tarX_indices):
    idx = tarX_indices[0].astype(jnp.int32).reshape(NW, PER_W)
    idx = jnp.pad(idx, ((0, 0), (0, IDX_ROWS * CHUNK - PER_W)))
    return _unpool(X, idx.reshape(NW, IDX_ROWS, CHUNK))


# speedup vs baseline: 1.3583x; 1.3583x over previous
import functools

import jax
import jax.numpy as jnp
from jax import lax
from jax.experimental import pallas as pl
from jax.experimental.pallas import tpu as pltpu
from jax.experimental.pallas import tpu_sc as plsc

N_NODES = 10000
NNZ = 320000
D_FEAT = 128

NUM_CORES = 2
NUM_SUBCORES = 16
NW = NUM_CORES * NUM_SUBCORES
PER_W = NNZ // NW
CHUNK = 72
N_FULL_CHUNKS = PER_W // CHUNK
TAIL_ROWS = PER_W - N_FULL_CHUNKS * CHUNK
NBUF = 4
N_ROUNDS = 33
N_PEEL = N_FULL_CHUNKS - NBUF * (N_ROUNDS + 1)
STAGE_ROWS = 624
STAGE_LAST = N_NODES - 15 * STAGE_ROWS


@functools.partial(
    pl.kernel,
    out_type=jax.ShapeDtypeStruct((NNZ, D_FEAT), jnp.float32),
    mesh=plsc.VectorSubcoreMesh(core_axis_name="c", subcore_axis_name="s"),
    scratch_types=[
        pltpu.VMEM_SHARED((N_NODES, D_FEAT), jnp.float32),
        pltpu.VMEM((PER_W,), jnp.int32),
        pltpu.VMEM((CHUNK, D_FEAT), jnp.float32),
        pltpu.VMEM((CHUNK, D_FEAT), jnp.float32),
        pltpu.VMEM((CHUNK, D_FEAT), jnp.float32),
        pltpu.VMEM((CHUNK, D_FEAT), jnp.float32),
        pltpu.SemaphoreType.DMA,
        pltpu.SemaphoreType.DMA,
        pltpu.SemaphoreType.DMA,
        pltpu.SemaphoreType.DMA,
    ],
)
def _unpool(
    x_hbm, idx_hbm, out_hbm, x_spmem, idx_v,
    r0, r1, r2, r3, s0, s1, s2, s3,
):
    bufs = (r0, r1, r2, r3)
    sems = (s0, s1, s2, s3)
    sid = lax.axis_index("s").astype(jnp.int32)
    wid = sid * jnp.int32(NUM_CORES) + lax.axis_index("c").astype(jnp.int32)
    base = wid * jnp.int32(PER_W)

    xoff = sid * jnp.int32(STAGE_ROWS)

    @pl.when(sid < jnp.int32(NUM_SUBCORES - 1))
    def _():
        pltpu.sync_copy(
            x_hbm.at[pl.ds(xoff, STAGE_ROWS)],
            x_spmem.at[pl.ds(xoff, STAGE_ROWS)],
        )

    @pl.when(sid == jnp.int32(NUM_SUBCORES - 1))
    def _():
        pltpu.sync_copy(
            x_hbm.at[pl.ds(xoff, STAGE_LAST)],
            x_spmem.at[pl.ds(xoff, STAGE_LAST)],
        )

    pltpu.sync_copy(idx_hbm.at[wid], idx_v)
    plsc.subcore_barrier()

    def start_gather(c, b):
        pltpu.async_copy(
            x_spmem.at[idx_v.at[pl.ds(c * jnp.int32(CHUNK), CHUNK)]],
            bufs[b],
            sems[b],
        )

    def wait_buf_dma(b):
        pltpu.make_async_copy(
            out_hbm.at[pl.ds(jnp.int32(0), CHUNK)], bufs[b], sems[b]
        ).wait()

    def start_store(c, b):
        off = base + c * jnp.int32(CHUNK)
        pltpu.async_copy(bufs[b], out_hbm.at[pl.ds(off, CHUNK)], sems[b])

    for b in range(NBUF):
        start_gather(jnp.int32(b), b)
    for b in range(NBUF):
        wait_buf_dma(b)
        start_store(jnp.int32(b), b)

    def body(g, _):
        for b in range(NBUF):
            c = g * jnp.int32(NBUF) + jnp.int32(b)
            wait_buf_dma(b)
            start_gather(c, b)
        for b in range(NBUF):
            c = g * jnp.int32(NBUF) + jnp.int32(b)
            wait_buf_dma(b)
            start_store(c, b)
        return ()

    lax.fori_loop(
        jnp.int32(1), jnp.int32(N_ROUNDS + 1), body, (), unroll=False
    )

    for t in range(N_PEEL):
        wait_buf_dma(t)
        start_gather(jnp.int32(NBUF * (N_ROUNDS + 1) + t), t)
    tb = N_PEEL
    tail_ioff = jnp.int32(N_FULL_CHUNKS * CHUNK)
    wait_buf_dma(tb)
    pltpu.async_copy(
        x_spmem.at[idx_v.at[pl.ds(tail_ioff, TAIL_ROWS)]],
        bufs[tb].at[pl.ds(jnp.int32(0), TAIL_ROWS)],
        sems[tb],
    )
    for t in range(N_PEEL):
        wait_buf_dma(t)
        start_store(jnp.int32(NBUF * (N_ROUNDS + 1) + t), t)
    tail_dst = out_hbm.at[pl.ds(base + tail_ioff, TAIL_ROWS)]
    pltpu.make_async_copy(
        out_hbm.at[pl.ds(jnp.int32(0), TAIL_ROWS)],
        bufs[tb].at[pl.ds(jnp.int32(0), TAIL_ROWS)],
        sems[tb],
    ).wait()
    pltpu.async_copy(
        bufs[tb].at[pl.ds(jnp.int32(0), TAIL_ROWS)], tail_dst, sems[tb]
    )
    pltpu.make_async_copy(
        bufs[tb].at[pl.ds(jnp.int32(0), TAIL_ROWS)], tail_dst, sems[tb]
    ).wait()
    for t in range(N_PEEL):
        wait_buf_dma(t)
    wait_buf_dma(NBUF - 1)


def kernel(X, tarX_indices):
    idx = tarX_indices[0].astype(jnp.int32).reshape(NW, PER_W)
    return _unpool(X, idx)
